# baseline (speedup 1.0000x reference)
import jax
import jax.numpy as jnp
from jax.experimental import pallas as pl
from jax.experimental.pallas import tpu as pltpu

Z, H, W = 8, 100, 88
N, T = 12000, 35
HP, WP = H + 2, W + 8
NYB = 17
BH = HP // NYB
GRID_R = 78464
TRASH = HP * WP * Z
N_PAD = 12288
BN = 128


def _relu(v):
    return jnp.maximum(v, 0.0)


def _vfe_body(x_ref, w1_ref, b1_ref, w2a_ref, w2b_ref, b2_ref,
              wfa_ref, wfb_ref, bf_ref, out_ref):
    x = x_ref[...]
    x2 = x.reshape(BN * T, 7)
    h1 = _relu(jnp.dot(x2, w1_ref[...], preferred_element_type=jnp.float32)
               + b1_ref[...])
    m1 = jnp.max(h1.reshape(BN, T, 16), axis=1)
    t2 = jnp.dot(m1, w2b_ref[...], preferred_element_type=jnp.float32) \
        + b2_ref[...]
    h2 = _relu(jnp.dot(h1, w2a_ref[...], preferred_element_type=jnp.float32)
               .reshape(BN, T, 64) + t2[:, None, :])
    m2 = jnp.max(h2, axis=1)
    tf = jnp.dot(m2, wfb_ref[...], preferred_element_type=jnp.float32) \
        + bf_ref[...]
    h3 = _relu(jnp.dot(h2.reshape(BN * T, 64), wfa_ref[...],
                       preferred_element_type=jnp.float32)
               .reshape(BN, T, 128) + tf[:, None, :])
    out_ref[...] = jnp.max(h3, axis=1)


def _vfe(x_pad, w1, b1, w2a, w2b, b2, wfa, wfb, bf):
    full = lambda *s: pl.BlockSpec(s, lambda i: (0,) * len(s))
    return pl.pallas_call(
        _vfe_body,
        grid=(N_PAD // BN,),
        in_specs=[
            pl.BlockSpec((BN, T, 7), lambda i: (i, 0, 0)),
            full(7, 16), full(16), full(16, 64), full(16, 64), full(64),
            full(64, 128), full(64, 128), full(128),
        ],
        out_specs=pl.BlockSpec((BN, 128), lambda i: (i, 0)),
        out_shape=jax.ShapeDtypeStruct((N_PAD, 128), jnp.float32),
    )(x_pad, w1, b1, w2a, w2b, b2, wfa, wfb, bf)


def _scatter_dense(h_pad, idx):
    n = jnp.arange(N_PAD, dtype=jnp.int32)
    y = jnp.pad(idx[:, 1], (0, N_PAD - N))
    xx = jnp.pad(idx[:, 2], (0, N_PAD - N))
    zz = jnp.pad(idx[:, 3], (0, N_PAD - N))
    cells = jnp.where(n < N, ((y + 1) * WP + (xx + 1)) * Z + zz, TRASH)
    owner = jnp.zeros((GRID_R,), jnp.int32).at[cells].max(n + 1)
    fidx = jnp.where(owner[cells] == n + 1, cells, TRASH)
    grid = jnp.zeros((GRID_R, 128), jnp.float32).at[fidx].set(h_pad)
    return grid


def _shift_cols(prod, dx):
    p = jnp.pad(prod, ((0, 0), (1, 1), (0, 0)))
    return p[:, dx:dx + WP, :]


def _border_mask(yb):
    ypad = yb * BH + jax.lax.broadcasted_iota(jnp.int32, (BH, WP, 1), 0)
    xpad = jax.lax.broadcasted_iota(jnp.int32, (BH, WP, 1), 1)
    ok = (ypad >= 1) & (ypad <= H) & (xpad >= 1) & (xpad <= W)
    return ok.astype(jnp.float32)


def _conv1_body(prev_ref, cur_ref, next_ref, w_ref, b_ref, out_ref):
    yb = pl.program_id(0)
    xs = [r[...].reshape(BH, WP, Z, 128) for r in (prev_ref, cur_ref, next_ref)]
    x = jnp.concatenate(xs, axis=0)
    zsl = [x[:, :, z, :] for z in range(Z)]
    zeros = jnp.zeros_like(zsl[0])
    mask = _border_mask(yb)
    for d in range(4):
        lo = 2 * d - 1
        x3 = jnp.concatenate(
            [zsl[lo] if lo >= 0 else zeros, zsl[lo + 1], zsl[lo + 2]], axis=-1)
        acc = jnp.zeros((BH, WP, 64), jnp.float32)
        for dy in range(3):
            rows = x3[BH - 1 + dy:2 * BH - 1 + dy]
            for dx in range(3):
                prod = jax.lax.dot_general(
                    rows, w_ref[dy * 3 + dx], (((2,), (0,)), ((), ())),
                    preferred_element_type=jnp.float32)
                acc = acc + _shift_cols(prod, dx)
        out_ref[d] = _relu(acc + b_ref[...]) * mask


def _conv1(grid, w3, bc1):
    blk = BH * WP * Z
    return pl.pallas_call(
        _conv1_body,
        grid=(NYB,),
        in_specs=[
            pl.BlockSpec((blk, 128), lambda yb: (jnp.maximum(yb - 1, 0), 0)),
            pl.BlockSpec((blk, 128), lambda yb: (yb, 0)),
            pl.BlockSpec((blk, 128), lambda yb: (jnp.minimum(yb + 1, NYB - 1), 0)),
            pl.BlockSpec((9, 384, 64), lambda yb: (0, 0, 0)),
            pl.BlockSpec((64,), lambda yb: (0,)),
        ],
        out_specs=pl.BlockSpec((4, BH, WP, 64), lambda yb: (0, yb, 0, 0)),
        out_shape=jax.ShapeDtypeStruct((4, HP, WP, 64), jnp.float32),
    )(grid, grid, grid, w3, bc1)


def _conv2_body(prev_ref, cur_ref, next_ref, w_ref, b_ref, out_ref):
    yb = pl.program_id(0)
    zsl = [jnp.concatenate([prev_ref[z], cur_ref[z], next_ref[z]], axis=0)
           for z in range(4)]
    zeros = jnp.zeros_like(zsl[0])
    mask = _border_mask(yb)
    accs = []
    for e in range(2):
        lo = 2 * e - 1
        x3 = jnp.concatenate(
            [zsl[lo] if lo >= 0 else zeros, zsl[lo + 1], zsl[lo + 2]], axis=-1)
        acc = jnp.zeros((BH, WP, 64), jnp.float32)
        for dy in range(3):
            rows = x3[BH - 1 + dy:2 * BH - 1 + dy]
            for dx in range(3):
                prod = jax.lax.dot_general(
                    rows, w_ref[dy * 3 + dx], (((2,), (0,)), ((), ())),
                    preferred_element_type=jnp.float32)
                acc = acc + _shift_cols(prod, dx)
        accs.append(_relu(acc + b_ref[...]))
    out_ref[...] = jnp.concatenate(accs, axis=-1) * mask


def _conv2(c1, w3, bc2):
    return pl.pallas_call(
        _conv2_body,
        grid=(NYB,),
        in_specs=[
            pl.BlockSpec((4, BH, WP, 64),
                         lambda yb: (0, jnp.maximum(yb - 1, 0), 0, 0)),
            pl.BlockSpec((4, BH, WP, 64), lambda yb: (0, yb, 0, 0)),
            pl.BlockSpec((4, BH, WP, 64),
                         lambda yb: (0, jnp.minimum(yb + 1, NYB - 1), 0, 0)),
            pl.BlockSpec((9, 192, 64), lambda yb: (0, 0, 0)),
            pl.BlockSpec((64,), lambda yb: (0,)),
        ],
        out_specs=pl.BlockSpec((BH, WP, 128), lambda yb: (yb, 0, 0)),
        out_shape=jax.ShapeDtypeStruct((HP, WP, 128), jnp.float32),
    )(c1, c1, c1, w3, bc2)


def _rpn_body(prev_ref, cur_ref, next_ref, wr_ref, br_ref, wh_ref, bh_ref,
              out_ref):
    x = jnp.concatenate([prev_ref[...], cur_ref[...], next_ref[...]], axis=0)
    acc = jnp.zeros((BH, WP, 128), jnp.float32)
    for dy in range(3):
        rows = x[BH - 1 + dy:2 * BH - 1 + dy]
        for dx in range(3):
            prod = jax.lax.dot_general(
                rows, wr_ref[dy * 3 + dx], (((2,), (0,)), ((), ())),
                preferred_element_type=jnp.float32)
            acc = acc + _shift_cols(prod, dx)
    r = _relu(acc + br_ref[...])
    out_ref[...] = (jnp.dot(r.reshape(BH * WP, 128), wh_ref[...],
                            preferred_element_type=jnp.float32)
                    + bh_ref[...]).reshape(BH, WP, 20)


def _rpn(cat, wrt, br1, wh, bh):
    return pl.pallas_call(
        _rpn_body,
        grid=(NYB,),
        in_specs=[
            pl.BlockSpec((BH, WP, 128), lambda yb: (jnp.maximum(yb - 1, 0), 0, 0)),
            pl.BlockSpec((BH, WP, 128), lambda yb: (yb, 0, 0)),
            pl.BlockSpec((BH, WP, 128),
                         lambda yb: (jnp.minimum(yb + 1, NYB - 1), 0, 0)),
            pl.BlockSpec((9, 128, 128), lambda yb: (0, 0, 0)),
            pl.BlockSpec((128,), lambda yb: (0,)),
            pl.BlockSpec((128, 20), lambda yb: (0, 0)),
            pl.BlockSpec((20,), lambda yb: (0,)),
        ],
        out_specs=pl.BlockSpec((BH, WP, 20), lambda yb: (yb, 0, 0)),
        out_shape=jax.ShapeDtypeStruct((HP, WP, 20), jnp.float32),
    )(cat, cat, cat, wrt, br1, wh, bh)


_PERM = tuple(2 * (j % 64) + (j // 64) for j in range(128))


def kernel(x, idx, w_vfe1, b_vfe1, w_vfe2, b_vfe2, w_fcn, b_fcn,
           wc1, bc1, wc2, bc2, wr1, br1, ws, bs, wreg, breg, wd, bd):
    x_pad = jnp.pad(x, ((0, N_PAD - N), (0, 0), (0, 0)))
    h_pad = _vfe(x_pad, w_vfe1, b_vfe1, w_vfe2[:16], w_vfe2[16:], b_vfe2,
                 w_fcn[:64], w_fcn[64:], b_fcn)

    grid = _scatter_dense(h_pad, idx)

    w1t = wc1.transpose(3, 4, 2, 1, 0).reshape(9, 384, 64)
    c1 = _conv1(grid, w1t, bc1)

    w2t = wc2.transpose(3, 4, 2, 1, 0).reshape(9, 192, 64)
    cat = _conv2(c1, w2t, bc2)

    wrt = wr1.transpose(2, 3, 1, 0)[:, :, _PERM, :].reshape(9, 128, 128)
    wh = jnp.concatenate([ws.reshape(2, 128).T, wreg.reshape(14, 128).T,
                          wd.reshape(4, 128).T], axis=1)
    bh = jnp.concatenate([bs, breg, bd])
    out20 = _rpn(cat, wrt, br1, wh, bh)

    o = out20[1:H + 1, 1:W + 1].transpose(2, 0, 1)[None]
    return (o[:, :2], o[:, 2:16], o[:, 16:20])


# speedup vs baseline: 2.0842x; 2.0842x over previous
import jax
import jax.numpy as jnp
from jax.experimental import pallas as pl
from jax.experimental.pallas import tpu as pltpu

Z, H, W = 8, 100, 88
N, T = 12000, 35
TP = 40
HP, WP = H + 2, W + 8
NYB = 17
BH = HP // NYB
GRID_R = 78464
TRASH = HP * WP * Z
N_PAD = 12288
BN = 128


def _relu(v):
    return jnp.maximum(v, 0.0)


def _vfe_body(x_ref, w1_ref, b1_ref, w2a_ref, w2b_ref, b2_ref,
              wfa_ref, wfb_ref, bf_ref, out_ref):
    x = x_ref[...]
    x2 = x.reshape(BN * TP, 7)
    h1 = _relu(jnp.dot(x2, w1_ref[...], preferred_element_type=jnp.float32)
               + b1_ref[...])
    m1 = jnp.max(h1.reshape(BN, TP, 16), axis=1)
    t2 = jnp.dot(m1, w2b_ref[...], preferred_element_type=jnp.float32) \
        + b2_ref[...]
    h2 = _relu(jnp.dot(h1, w2a_ref[...], preferred_element_type=jnp.float32)
               .reshape(BN, TP, 64) + t2[:, None, :])
    m2 = jnp.max(h2, axis=1)
    tf = jnp.dot(m2, wfb_ref[...], preferred_element_type=jnp.float32) \
        + bf_ref[...]
    h3 = _relu(jnp.dot(h2.reshape(BN * TP, 64), wfa_ref[...],
                       preferred_element_type=jnp.float32)
               .reshape(BN, TP, 128) + tf[:, None, :])
    out_ref[...] = jnp.max(h3, axis=1)


def _vfe(x_pad, w1, b1, w2a, w2b, b2, wfa, wfb, bf):
    full = lambda *s: pl.BlockSpec(s, lambda i: (0,) * len(s))
    return pl.pallas_call(
        _vfe_body,
        grid=(N_PAD // BN,),
        in_specs=[
            pl.BlockSpec((BN, TP, 7), lambda i: (i, 0, 0)),
            full(7, 16), full(16), full(16, 64), full(16, 64), full(64),
            full(64, 128), full(64, 128), full(128),
        ],
        out_specs=pl.BlockSpec((BN, 128), lambda i: (i, 0)),
        out_shape=jax.ShapeDtypeStruct((N_PAD, 128), jnp.float32),
    )(x_pad, w1, b1, w2a, w2b, b2, wfa, wfb, bf)


def _scatter_dense(h_pad, idx):
    n = jnp.arange(N_PAD, dtype=jnp.int32)
    y = jnp.pad(idx[:, 1], (0, N_PAD - N))
    xx = jnp.pad(idx[:, 2], (0, N_PAD - N))
    zz = jnp.pad(idx[:, 3], (0, N_PAD - N))
    cells = jnp.where(n < N, ((y + 1) * WP + (xx + 1)) * Z + zz, TRASH)
    owner = jnp.zeros((GRID_R,), jnp.int32).at[cells].max(n + 1)
    fidx = jnp.where(owner[cells] == n + 1, cells, TRASH)
    grid = jnp.zeros((GRID_R, 128), jnp.float32).at[fidx].set(h_pad)
    return grid


def _shift_cols(prod, dx):
    p = jnp.pad(prod, ((0, 0), (1, 1), (0, 0)))
    return p[:, dx:dx + WP, :]


def _border_mask(yb):
    ypad = yb * BH + jax.lax.broadcasted_iota(jnp.int32, (BH, WP, 1), 0)
    xpad = jax.lax.broadcasted_iota(jnp.int32, (BH, WP, 1), 1)
    ok = (ypad >= 1) & (ypad <= H) & (xpad >= 1) & (xpad <= W)
    return ok.astype(jnp.float32)


def _conv1_body(prev_ref, cur_ref, next_ref, w_ref, b_ref, out_ref):
    yb = pl.program_id(0)
    xs = [r[...].reshape(BH, WP, Z, 128) for r in (prev_ref, cur_ref, next_ref)]
    x = jnp.concatenate(xs, axis=0)
    zsl = [x[:, :, z, :] for z in range(Z)]
    zeros = jnp.zeros_like(zsl[0])
    mask = _border_mask(yb)
    for d in range(4):
        lo = 2 * d - 1
        x3 = jnp.concatenate(
            [zsl[lo] if lo >= 0 else zeros, zsl[lo + 1], zsl[lo + 2]], axis=-1)
        acc = jnp.zeros((BH, WP, 64), jnp.float32)
        for dy in range(3):
            rows = x3[BH - 1 + dy:2 * BH - 1 + dy]
            for dx in range(3):
                prod = jax.lax.dot_general(
                    rows, w_ref[dy * 3 + dx], (((2,), (0,)), ((), ())),
                    preferred_element_type=jnp.float32)
                acc = acc + _shift_cols(prod, dx)
        out_ref[d] = _relu(acc + b_ref[...]) * mask


def _conv1(grid, w3, bc1):
    blk = BH * WP * Z
    return pl.pallas_call(
        _conv1_body,
        grid=(NYB,),
        in_specs=[
            pl.BlockSpec((blk, 128), lambda yb: (jnp.maximum(yb - 1, 0), 0)),
            pl.BlockSpec((blk, 128), lambda yb: (yb, 0)),
            pl.BlockSpec((blk, 128), lambda yb: (jnp.minimum(yb + 1, NYB - 1), 0)),
            pl.BlockSpec((9, 384, 64), lambda yb: (0, 0, 0)),
            pl.BlockSpec((64,), lambda yb: (0,)),
        ],
        out_specs=pl.BlockSpec((4, BH, WP, 64), lambda yb: (0, yb, 0, 0)),
        out_shape=jax.ShapeDtypeStruct((4, HP, WP, 64), jnp.float32),
    )(grid, grid, grid, w3, bc1)


def _conv2_body(prev_ref, cur_ref, next_ref, w_ref, b_ref, out_ref):
    yb = pl.program_id(0)
    zsl = [jnp.concatenate([prev_ref[z], cur_ref[z], next_ref[z]], axis=0)
           for z in range(4)]
    zeros = jnp.zeros_like(zsl[0])
    mask = _border_mask(yb)
    accs = []
    for e in range(2):
        lo = 2 * e - 1
        x3 = jnp.concatenate(
            [zsl[lo] if lo >= 0 else zeros, zsl[lo + 1], zsl[lo + 2]], axis=-1)
        acc = jnp.zeros((BH, WP, 64), jnp.float32)
        for dy in range(3):
            rows = x3[BH - 1 + dy:2 * BH - 1 + dy]
            for dx in range(3):
                prod = jax.lax.dot_general(
                    rows, w_ref[dy * 3 + dx], (((2,), (0,)), ((), ())),
                    preferred_element_type=jnp.float32)
                acc = acc + _shift_cols(prod, dx)
        accs.append(_relu(acc + b_ref[...]))
    out_ref[...] = jnp.concatenate(accs, axis=-1) * mask


def _conv2(c1, w3, bc2):
    return pl.pallas_call(
        _conv2_body,
        grid=(NYB,),
        in_specs=[
            pl.BlockSpec((4, BH, WP, 64),
                         lambda yb: (0, jnp.maximum(yb - 1, 0), 0, 0)),
            pl.BlockSpec((4, BH, WP, 64), lambda yb: (0, yb, 0, 0)),
            pl.BlockSpec((4, BH, WP, 64),
                         lambda yb: (0, jnp.minimum(yb + 1, NYB - 1), 0, 0)),
            pl.BlockSpec((9, 192, 64), lambda yb: (0, 0, 0)),
            pl.BlockSpec((64,), lambda yb: (0,)),
        ],
        out_specs=pl.BlockSpec((BH, WP, 128), lambda yb: (yb, 0, 0)),
        out_shape=jax.ShapeDtypeStruct((HP, WP, 128), jnp.float32),
    )(c1, c1, c1, w3, bc2)


def _rpn_body(prev_ref, cur_ref, next_ref, wr_ref, br_ref, wh_ref, bh_ref,
              out_ref):
    x = jnp.concatenate([prev_ref[...], cur_ref[...], next_ref[...]], axis=0)
    acc = jnp.zeros((BH, WP, 128), jnp.float32)
    for dy in range(3):
        rows = x[BH - 1 + dy:2 * BH - 1 + dy]
        for dx in range(3):
            prod = jax.lax.dot_general(
                rows, wr_ref[dy * 3 + dx], (((2,), (0,)), ((), ())),
                preferred_element_type=jnp.float32)
            acc = acc + _shift_cols(prod, dx)
    r = _relu(acc + br_ref[...])
    out_ref[...] = (jnp.dot(r.reshape(BH * WP, 128), wh_ref[...],
                            preferred_element_type=jnp.float32)
                    + bh_ref[...]).reshape(BH, WP, 20)


def _rpn(cat, wrt, br1, wh, bh):
    return pl.pallas_call(
        _rpn_body,
        grid=(NYB,),
        in_specs=[
            pl.BlockSpec((BH, WP, 128), lambda yb: (jnp.maximum(yb - 1, 0), 0, 0)),
            pl.BlockSpec((BH, WP, 128), lambda yb: (yb, 0, 0)),
            pl.BlockSpec((BH, WP, 128),
                         lambda yb: (jnp.minimum(yb + 1, NYB - 1), 0, 0)),
            pl.BlockSpec((9, 128, 128), lambda yb: (0, 0, 0)),
            pl.BlockSpec((128,), lambda yb: (0,)),
            pl.BlockSpec((128, 20), lambda yb: (0, 0)),
            pl.BlockSpec((20,), lambda yb: (0,)),
        ],
        out_specs=pl.BlockSpec((BH, WP, 20), lambda yb: (yb, 0, 0)),
        out_shape=jax.ShapeDtypeStruct((HP, WP, 20), jnp.float32),
    )(cat, cat, cat, wrt, br1, wh, bh)


_PERM = tuple(2 * (j % 64) + (j // 64) for j in range(128))


def kernel(x, idx, w_vfe1, b_vfe1, w_vfe2, b_vfe2, w_fcn, b_fcn,
           wc1, bc1, wc2, bc2, wr1, br1, ws, bs, wreg, breg, wd, bd):
    xt = jnp.concatenate([x, jnp.broadcast_to(x[:, :1], (N, TP - T, 7))], axis=1)
    x_pad = jnp.pad(xt, ((0, N_PAD - N), (0, 0), (0, 0)))
    h_pad = _vfe(x_pad, w_vfe1, b_vfe1, w_vfe2[:16], w_vfe2[16:], b_vfe2,
                 w_fcn[:64], w_fcn[64:], b_fcn)

    grid = _scatter_dense(h_pad, idx)

    w1t = wc1.transpose(3, 4, 2, 1, 0).reshape(9, 384, 64)
    c1 = _conv1(grid, w1t, bc1)

    w2t = wc2.transpose(3, 4, 2, 1, 0).reshape(9, 192, 64)
    cat = _conv2(c1, w2t, bc2)

    wrt = wr1.transpose(2, 3, 1, 0)[:, :, _PERM, :].reshape(9, 128, 128)
    wh = jnp.concatenate([ws.reshape(2, 128).T, wreg.reshape(14, 128).T,
                          wd.reshape(4, 128).T], axis=1)
    bh = jnp.concatenate([bs, breg, bd])
    out20 = _rpn(cat, wrt, br1, wh, bh)

    o = out20[1:H + 1, 1:W + 1].transpose(2, 0, 1)[None]
    return (o[:, :2], o[:, 2:16], o[:, 16:20])


# speedup vs baseline: 2.1989x; 1.0550x over previous
import jax
import jax.numpy as jnp
from jax.experimental import pallas as pl
from jax.experimental.pallas import tpu as pltpu

Z, H, W = 8, 100, 88
N, T = 12000, 35
TP = 40
HP, WP = H + 2, W + 8
NYB = 17
BH = HP // NYB
GRID_R = 78464
TRASH = HP * WP * Z
N_PAD = 12288
BN = 256


def _relu(v):
    return jnp.maximum(v, 0.0)


def _vfe_body(x_ref, w1_ref, b1_ref, w2a_ref, w2b_ref, b2_ref,
              wfa_ref, wfb_ref, bf_ref, out_ref):
    x = x_ref[...]
    x2 = x.reshape(BN * TP, 7)
    h1 = _relu(jnp.dot(x2, w1_ref[...], preferred_element_type=jnp.float32)
               + b1_ref[...])
    m1 = jnp.max(h1.reshape(BN, TP, 16), axis=1)
    t2 = jnp.dot(m1, w2b_ref[...], preferred_element_type=jnp.float32) \
        + b2_ref[...]
    h2 = _relu(jnp.dot(h1, w2a_ref[...], preferred_element_type=jnp.float32)
               .reshape(BN, TP, 64) + t2[:, None, :])
    m2 = jnp.max(h2, axis=1)
    tf = jnp.dot(m2, wfb_ref[...], preferred_element_type=jnp.float32) \
        + bf_ref[...]
    h3 = _relu(jnp.dot(h2.reshape(BN * TP, 64), wfa_ref[...],
                       preferred_element_type=jnp.float32)
               .reshape(BN, TP, 128) + tf[:, None, :])
    out_ref[...] = jnp.max(h3, axis=1)


def _vfe(x_pad, w1, b1, w2a, w2b, b2, wfa, wfb, bf):
    full = lambda *s: pl.BlockSpec(s, lambda i: (0,) * len(s))
    return pl.pallas_call(
        _vfe_body,
        grid=(N_PAD // BN,),
        in_specs=[
            pl.BlockSpec((BN, TP, 7), lambda i: (i, 0, 0)),
            full(7, 16), full(16), full(16, 64), full(16, 64), full(64),
            full(64, 128), full(64, 128), full(128),
        ],
        out_specs=pl.BlockSpec((BN, 128), lambda i: (i, 0)),
        out_shape=jax.ShapeDtypeStruct((N_PAD, 128), jnp.float32),
    )(x_pad, w1, b1, w2a, w2b, b2, wfa, wfb, bf)


def _scatter_dense(h_pad, idx):
    n = jnp.arange(N_PAD, dtype=jnp.int32)
    y = jnp.pad(idx[:, 1], (0, N_PAD - N))
    xx = jnp.pad(idx[:, 2], (0, N_PAD - N))
    zz = jnp.pad(idx[:, 3], (0, N_PAD - N))
    cells = jnp.where(n < N, ((y + 1) * WP + (xx + 1)) * Z + zz, TRASH)
    owner = jnp.zeros((GRID_R,), jnp.int32).at[cells].max(n + 1)
    fidx = jnp.where(owner[cells] == n + 1, cells, TRASH)
    grid = jnp.zeros((GRID_R, 128), jnp.float32).at[fidx].set(h_pad)
    return grid


def _shift_cols(prod, dx):
    p = jnp.pad(prod, ((0, 0), (1, 1), (0, 0)))
    return p[:, dx:dx + WP, :]


def _border_mask(yb):
    ypad = yb * BH + jax.lax.broadcasted_iota(jnp.int32, (BH, WP, 1), 0)
    xpad = jax.lax.broadcasted_iota(jnp.int32, (BH, WP, 1), 1)
    ok = (ypad >= 1) & (ypad <= H) & (xpad >= 1) & (xpad <= W)
    return ok.astype(jnp.float32)


def _conv1_body(prev_ref, cur_ref, next_ref, w_ref, b_ref, out_ref):
    yb = pl.program_id(0)
    x = jnp.concatenate(
        [r[...].reshape(BH, WP, Z * 128) for r in (prev_ref, cur_ref, next_ref)],
        axis=0)
    mask = _border_mask(yb)
    acc = jnp.zeros((BH, WP, 256), jnp.float32)
    for dy in range(3):
        rows = x[BH - 1 + dy:2 * BH - 1 + dy]
        for dx in range(3):
            prod = jax.lax.dot_general(
                rows, w_ref[dy * 3 + dx], (((2,), (0,)), ((), ())),
                preferred_element_type=jnp.float32)
            acc = acc + _shift_cols(prod, dx)
    out_ref[...] = _relu(acc + b_ref[...]) * mask


def _conv1(grid, w3, bc1):
    blk = BH * WP * Z
    return pl.pallas_call(
        _conv1_body,
        grid=(NYB,),
        in_specs=[
            pl.BlockSpec((blk, 128), lambda yb: (jnp.maximum(yb - 1, 0), 0)),
            pl.BlockSpec((blk, 128), lambda yb: (yb, 0)),
            pl.BlockSpec((blk, 128), lambda yb: (jnp.minimum(yb + 1, NYB - 1), 0)),
            pl.BlockSpec((9, 1024, 256), lambda yb: (0, 0, 0)),
            pl.BlockSpec((256,), lambda yb: (0,)),
        ],
        out_specs=pl.BlockSpec((BH, WP, 256), lambda yb: (yb, 0, 0)),
        out_shape=jax.ShapeDtypeStruct((HP, WP, 256), jnp.float32),
    )(grid, grid, grid, w3, bc1)


def _conv2_body(prev_ref, cur_ref, next_ref, w_ref, b_ref, out_ref):
    yb = pl.program_id(0)
    x = jnp.concatenate([prev_ref[...], cur_ref[...], next_ref[...]], axis=0)
    mask = _border_mask(yb)
    acc = jnp.zeros((BH, WP, 128), jnp.float32)
    for dy in range(3):
        rows = x[BH - 1 + dy:2 * BH - 1 + dy]
        for dx in range(3):
            prod = jax.lax.dot_general(
                rows, w_ref[dy * 3 + dx], (((2,), (0,)), ((), ())),
                preferred_element_type=jnp.float32)
            acc = acc + _shift_cols(prod, dx)
    out_ref[...] = _relu(acc + b_ref[...]) * mask


def _conv2(c1, w3, bc2):
    return pl.pallas_call(
        _conv2_body,
        grid=(NYB,),
        in_specs=[
            pl.BlockSpec((BH, WP, 256), lambda yb: (jnp.maximum(yb - 1, 0), 0, 0)),
            pl.BlockSpec((BH, WP, 256), lambda yb: (yb, 0, 0)),
            pl.BlockSpec((BH, WP, 256), lambda yb: (jnp.minimum(yb + 1, NYB - 1), 0, 0)),
            pl.BlockSpec((9, 256, 128), lambda yb: (0, 0, 0)),
            pl.BlockSpec((128,), lambda yb: (0,)),
        ],
        out_specs=pl.BlockSpec((BH, WP, 128), lambda yb: (yb, 0, 0)),
        out_shape=jax.ShapeDtypeStruct((HP, WP, 128), jnp.float32),
    )(c1, c1, c1, w3, bc2)


def _rpn_body(prev_ref, cur_ref, next_ref, wr_ref, br_ref, wh_ref, bh_ref,
              out_ref):
    x = jnp.concatenate([prev_ref[...], cur_ref[...], next_ref[...]], axis=0)
    acc = jnp.zeros((BH, WP, 128), jnp.float32)
    for dy in range(3):
        rows = x[BH - 1 + dy:2 * BH - 1 + dy]
        for dx in range(3):
            prod = jax.lax.dot_general(
                rows, wr_ref[dy * 3 + dx], (((2,), (0,)), ((), ())),
                preferred_element_type=jnp.float32)
            acc = acc + _shift_cols(prod, dx)
    r = _relu(acc + br_ref[...])
    out_ref[...] = (jnp.dot(r.reshape(BH * WP, 128), wh_ref[...],
                            preferred_element_type=jnp.float32)
                    + bh_ref[...]).reshape(BH, WP, 20)


def _rpn(cat, wrt, br1, wh, bh):
    return pl.pallas_call(
        _rpn_body,
        grid=(NYB,),
        in_specs=[
            pl.BlockSpec((BH, WP, 128), lambda yb: (jnp.maximum(yb - 1, 0), 0, 0)),
            pl.BlockSpec((BH, WP, 128), lambda yb: (yb, 0, 0)),
            pl.BlockSpec((BH, WP, 128),
                         lambda yb: (jnp.minimum(yb + 1, NYB - 1), 0, 0)),
            pl.BlockSpec((9, 128, 128), lambda yb: (0, 0, 0)),
            pl.BlockSpec((128,), lambda yb: (0,)),
            pl.BlockSpec((128, 20), lambda yb: (0, 0)),
            pl.BlockSpec((20,), lambda yb: (0,)),
        ],
        out_specs=pl.BlockSpec((BH, WP, 20), lambda yb: (yb, 0, 0)),
        out_shape=jax.ShapeDtypeStruct((HP, WP, 20), jnp.float32),
    )(cat, cat, cat, wrt, br1, wh, bh)


_PERM = tuple(2 * (j % 64) + (j // 64) for j in range(128))


def kernel(x, idx, w_vfe1, b_vfe1, w_vfe2, b_vfe2, w_fcn, b_fcn,
           wc1, bc1, wc2, bc2, wr1, br1, ws, bs, wreg, breg, wd, bd):
    xt = jnp.concatenate([x, jnp.broadcast_to(x[:, :1], (N, TP - T, 7))], axis=1)
    x_pad = jnp.pad(xt, ((0, N_PAD - N), (0, 0), (0, 0)))
    h_pad = _vfe(x_pad, w_vfe1, b_vfe1, w_vfe2[:16], w_vfe2[16:], b_vfe2,
                 w_fcn[:64], w_fcn[64:], b_fcn)

    grid = _scatter_dense(h_pad, idx)

    wt = wc1.transpose(3, 4, 2, 1, 0)
    w1t = jnp.zeros((3, 3, Z, 128, 4, 64), jnp.float32)
    for d in range(4):
        for dz in range(3):
            z = 2 * d - 1 + dz
            if 0 <= z < Z:
                w1t = w1t.at[:, :, z, :, d, :].set(wt[:, :, dz])
    w1t = w1t.reshape(9, Z * 128, 256)
    c1 = _conv1(grid, w1t, jnp.tile(bc1, 4))

    wt2 = wc2.transpose(3, 4, 2, 1, 0)
    w2t = jnp.zeros((3, 3, 4, 64, 2, 64), jnp.float32)
    for e in range(2):
        for dz in range(3):
            z = 2 * e - 1 + dz
            if 0 <= z < 4:
                w2t = w2t.at[:, :, z, :, e, :].set(wt2[:, :, dz])
    w2t = w2t.reshape(9, 256, 128)
    cat = _conv2(c1, w2t, jnp.tile(bc2, 2))

    wrt = wr1.transpose(2, 3, 1, 0)[:, :, _PERM, :].reshape(9, 128, 128)
    wh = jnp.concatenate([ws.reshape(2, 128).T, wreg.reshape(14, 128).T,
                          wd.reshape(4, 128).T], axis=1)
    bh = jnp.concatenate([bs, breg, bd])
    out20 = _rpn(cat, wrt, br1, wh, bh)

    o = out20[1:H + 1, 1:W + 1].transpose(2, 0, 1)[None]
    return (o[:, :2], o[:, 2:16], o[:, 16:20])
